# baseline (speedup 1.0000x reference)
import functools

import jax
import jax.numpy as jnp
from jax import lax
from jax.experimental import pallas as pl
from jax.experimental.pallas import tpu as pltpu
from jax.experimental.pallas import tpu_sc as plsc

_NC, _NS, _L = 2, 16, 16
_NW = _NC * _NS
_UNROLL = 16
_CH = 16
_NBUF = 3


@functools.lru_cache(maxsize=None)
def _make_sc(B, S, D):
    s_per_w = S // _NW
    n_ch = s_per_w // _CH
    n_chunks = n_ch * B

    mesh = plsc.VectorSubcoreMesh(
        core_axis_name="c", subcore_axis_name="s",
        num_cores=_NC, num_subcores=_NS)

    @functools.partial(
        pl.kernel, mesh=mesh,
        out_type=jax.ShapeDtypeStruct((B, S, D), jnp.float32),
        scratch_types=(
            [pltpu.VMEM((s_per_w, D), jnp.float32)]
            + [pltpu.VMEM((_CH, D), jnp.float32)] * _NBUF
            + [pltpu.SemaphoreType.DMA] * (1 + 2 * _NBUF)
        ),
    )
    def k(x_hbm, pe_hbm, out_hbm, pe_v, *rest):
        xv = rest[:_NBUF]
        pe_sem = rest[_NBUF]
        ins = rest[_NBUF + 1:_NBUF + 1 + _NBUF]
        outs = rest[_NBUF + 1 + _NBUF:]
        wid = lax.axis_index("s") * _NC + lax.axis_index("c")
        base_s = wid * s_per_w

        pe_d = pltpu.async_copy(
            pe_hbm.at[pl.ds(base_s, s_per_w)], pe_v, pe_sem)

        def start_in(t):
            j, b = divmod(t, B)
            return pltpu.async_copy(
                x_hbm.at[b, pl.ds(base_s + j * _CH, _CH)],
                xv[t % _NBUF], ins[t % _NBUF])

        def start_out(t):
            j, b = divmod(t, B)
            return pltpu.async_copy(
                xv[t % _NBUF],
                out_hbm.at[b, pl.ds(base_s + j * _CH, _CH)],
                outs[t % _NBUF])

        d_in = {t: start_in(t) for t in range(_NBUF - 1)}
        d_out = {}
        pe_d.wait()
        for t in range(n_chunks):
            ta = t + _NBUF - 1
            if ta < n_chunks:
                if ta - _NBUF >= 0:
                    d_out[ta - _NBUF].wait()
                d_in[ta] = start_in(ta)
            d_in[t].wait()
            j = t // B
            buf = xv[t % _NBUF]
            dshift = D.bit_length() - 1

            def body(i, buf=buf, j=j):
                r = i >> dshift
                c = pl.multiple_of(i & (D - 1), _L)
                plsc.addupdate(
                    buf.at[r, pl.ds(c, _L)],
                    pe_v[j * _CH + r, pl.ds(c, _L)])

            plsc.parallel_loop(0, _CH * D, _L, unroll=_UNROLL)(body)
            d_out[t] = start_out(t)
        for t in range(n_chunks - _NBUF, n_chunks):
            if t >= 0:
                d_out[t].wait()

    return k


def kernel(inputs, pos_embed):
    B, S, D = inputs.shape
    return _make_sc(B, S, D)(inputs, pos_embed)


# speedup vs baseline: 1.1099x; 1.1099x over previous
import functools

import jax
import jax.numpy as jnp
from jax import lax
from jax.experimental import pallas as pl
from jax.experimental.pallas import tpu as pltpu
from jax.experimental.pallas import tpu_sc as plsc

_NC, _NS, _L = 2, 16, 16
_NW = _NC * _NS
_UNROLL = 16
_CH = 16
_NBUF = 3
_S_SC = 512
_BS_TC = 256


@functools.lru_cache(maxsize=None)
def _make_sc(B, S_sc, D):
    s_per_w = S_sc // _NW
    ch = min(_CH, s_per_w)
    n_ch = s_per_w // ch
    n_chunks = n_ch * B

    mesh = plsc.VectorSubcoreMesh(
        core_axis_name="c", subcore_axis_name="s",
        num_cores=_NC, num_subcores=_NS)

    @functools.partial(
        pl.kernel, mesh=mesh,
        out_type=jax.ShapeDtypeStruct((B, S_sc, D), jnp.float32),
        scratch_types=(
            [pltpu.VMEM((s_per_w, D), jnp.float32)]
            + [pltpu.VMEM((ch, D), jnp.float32)] * _NBUF
            + [pltpu.SemaphoreType.DMA] * (1 + 2 * _NBUF)
        ),
    )
    def k(x_hbm, pe_hbm, out_hbm, pe_v, *rest):
        xv = rest[:_NBUF]
        pe_sem = rest[_NBUF]
        ins = rest[_NBUF + 1:_NBUF + 1 + _NBUF]
        outs = rest[_NBUF + 1 + _NBUF:]
        wid = lax.axis_index("s") * _NC + lax.axis_index("c")
        base_s = wid * s_per_w

        pe_d = pltpu.async_copy(
            pe_hbm.at[pl.ds(base_s, s_per_w)], pe_v, pe_sem)

        def start_in(t):
            j, b = divmod(t, B)
            return pltpu.async_copy(
                x_hbm.at[b, pl.ds(base_s + j * ch, ch)],
                xv[t % _NBUF], ins[t % _NBUF])

        def start_out(t):
            j, b = divmod(t, B)
            return pltpu.async_copy(
                xv[t % _NBUF],
                out_hbm.at[b, pl.ds(base_s + j * ch, ch)],
                outs[t % _NBUF])

        d_in = {t: start_in(t) for t in range(min(_NBUF - 1, n_chunks))}
        d_out = {}
        pe_d.wait()
        for t in range(n_chunks):
            ta = t + _NBUF - 1
            if ta < n_chunks:
                if ta - _NBUF >= 0:
                    d_out[ta - _NBUF].wait()
                d_in[ta] = start_in(ta)
            d_in[t].wait()
            j = t // B
            buf = xv[t % _NBUF]
            dshift = D.bit_length() - 1

            def body(i, buf=buf, j=j):
                r = i >> dshift
                c = pl.multiple_of(i & (D - 1), _L)
                plsc.addupdate(
                    buf.at[r, pl.ds(c, _L)],
                    pe_v[j * ch + r, pl.ds(c, _L)])

            plsc.parallel_loop(0, ch * D, _L, unroll=_UNROLL)(body)
            d_out[t] = start_out(t)
        for t in range(max(0, n_chunks - _NBUF), n_chunks):
            d_out[t].wait()

    return k


def _tc_add_body(x_ref, pe_ref, o_ref):
    o_ref[...] = x_ref[...] + pe_ref[...]


def _tc_call(inputs, pos_embed, s_start):
    B, S, D = inputs.shape
    blk0 = s_start // _BS_TC
    grid = ((S - s_start) // _BS_TC,)
    return pl.pallas_call(
        _tc_add_body,
        grid=grid,
        in_specs=[
            pl.BlockSpec((B, _BS_TC, D), lambda i: (0, i + blk0, 0)),
            pl.BlockSpec((1, _BS_TC, D), lambda i: (0, i + blk0, 0)),
        ],
        out_specs=pl.BlockSpec((B, _BS_TC, D), lambda i: (0, i + blk0, 0)),
        out_shape=jax.ShapeDtypeStruct((B, S, D), inputs.dtype),
    )(inputs, pos_embed[None])


def kernel(inputs, pos_embed):
    B, S, D = inputs.shape
    o_sc = _make_sc(B, _S_SC, D)(inputs, pos_embed)
    o_full = _tc_call(inputs, pos_embed, _S_SC)
    return lax.dynamic_update_slice(o_full, o_sc, (0, 0, 0))


# speedup vs baseline: 2.0259x; 1.8254x over previous
import jax
import jax.numpy as jnp
from jax.experimental import pallas as pl

_BS = 128


def _add_body(x_ref, pe_ref, o_ref):
    o_ref[...] = x_ref[...] + pe_ref[...]


def kernel(inputs, pos_embed):
    B, S, D = inputs.shape
    grid = (S // _BS,)
    return pl.pallas_call(
        _add_body,
        grid=grid,
        in_specs=[
            pl.BlockSpec((B, _BS, D), lambda i: (0, i, 0)),
            pl.BlockSpec((1, _BS, D), lambda i: (0, i, 0)),
        ],
        out_specs=pl.BlockSpec((B, _BS, D), lambda i: (0, i, 0)),
        out_shape=jax.ShapeDtypeStruct((B, S, D), inputs.dtype),
    )(inputs, pos_embed[None])


# speedup vs baseline: 2.1649x; 1.0686x over previous
import jax
import jax.numpy as jnp
from jax.experimental import pallas as pl

_BS = 512


def _add_body(x_ref, pe_ref, o_ref):
    o_ref[...] = x_ref[...] + pe_ref[...]


def kernel(inputs, pos_embed):
    B, S, D = inputs.shape
    grid = (S // _BS,)
    return pl.pallas_call(
        _add_body,
        grid=grid,
        in_specs=[
            pl.BlockSpec((B, _BS, D), lambda i: (0, i, 0)),
            pl.BlockSpec((1, _BS, D), lambda i: (0, i, 0)),
        ],
        out_specs=pl.BlockSpec((B, _BS, D), lambda i: (0, i, 0)),
        out_shape=jax.ShapeDtypeStruct((B, S, D), inputs.dtype),
    )(inputs, pos_embed[None])


# speedup vs baseline: 2.2672x; 1.0472x over previous
import jax
import jax.numpy as jnp
from jax.experimental import pallas as pl

_BS = 512


def _add_body(x_ref, pe_ref, o_ref):
    i = pl.program_id(0)
    o_ref[...] = x_ref[...] + pe_ref[:, pl.ds(i * _BS, _BS), :]


def kernel(inputs, pos_embed):
    B, S, D = inputs.shape
    grid = (S // _BS,)
    return pl.pallas_call(
        _add_body,
        grid=grid,
        in_specs=[
            pl.BlockSpec((B, _BS, D), lambda i: (0, i, 0)),
            pl.BlockSpec((1, S, D), lambda i: (0, 0, 0)),
        ],
        out_specs=pl.BlockSpec((B, _BS, D), lambda i: (0, i, 0)),
        out_shape=jax.ShapeDtypeStruct((B, S, D), inputs.dtype),
    )(inputs, pos_embed[None])
